# Initial kernel scaffold; baseline (speedup 1.0000x reference)
import jax
import jax.numpy as jnp
from jax.experimental import pallas as pl


def kernel(x, W_enc, b_enc, W_dec, b_dec):
    raise NotImplementedError("write your pallas kernel here")


# speedup vs baseline: 5.4849x; 5.4849x over previous
import jax
import jax.numpy as jnp
from jax.experimental import pallas as pl
from jax.experimental.pallas import tpu as pltpu

INPUT_DIM = 2048
LATENT_DIM = 16384
N_TOKENS = 8192
K = 32

R1 = 2048
L1 = 512


def _enc_kernel(x_ref, w_ref, b_ref, pre_ref):
    pre_ref[...] = jax.lax.dot_general(
        x_ref[...], w_ref[...], (((1,), (1,)), ((), ())),
        preferred_element_type=jnp.float32) + b_ref[...]


def _encode(x16, W16, b_enc2d):
    return pl.pallas_call(
        _enc_kernel,
        grid=(N_TOKENS // R1, LATENT_DIM // L1),
        in_specs=[
            pl.BlockSpec((R1, INPUT_DIM), lambda i, j: (i, 0)),
            pl.BlockSpec((L1, INPUT_DIM), lambda i, j: (j, 0)),
            pl.BlockSpec((1, L1), lambda i, j: (0, j)),
        ],
        out_specs=pl.BlockSpec((R1, L1), lambda i, j: (i, j)),
        out_shape=jax.ShapeDtypeStruct((N_TOKENS, LATENT_DIM), jnp.float32),
        compiler_params=pltpu.CompilerParams(
            dimension_semantics=("parallel", "parallel")),
    )(x16, W16, b_enc2d)


R2 = 256


def _thresh_kernel(pre_ref, tau_ref, u_s):
    u_s[...] = jax.lax.bitcast_convert_type(
        pre_ref[...], jnp.int32) & jnp.int32(0x7FFFFFFF)
    u = u_s[...]

    def body(i, t):
        cand = t | (jnp.int32(1) << (jnp.int32(30) - i))
        cnt = jnp.sum((u >= cand).astype(jnp.int32), axis=1, keepdims=True)
        return jnp.where(cnt >= K, cand, t)

    t = jax.lax.fori_loop(0, 31, body, jnp.zeros((R2, 1), jnp.int32))
    tau_ref[...] = jnp.broadcast_to(t, (R2, 128))


def _thresholds(pre):
    return pl.pallas_call(
        _thresh_kernel,
        grid=(N_TOKENS // R2,),
        in_specs=[pl.BlockSpec((R2, LATENT_DIM), lambda i: (i, 0))],
        out_specs=pl.BlockSpec((R2, 128), lambda i: (i, 0)),
        out_shape=jax.ShapeDtypeStruct((N_TOKENS, 128), jnp.int32),
        scratch_shapes=[pltpu.VMEM((R2, LATENT_DIM), jnp.int32)],
        compiler_params=pltpu.CompilerParams(
            dimension_semantics=("parallel",)),
    )(pre)


R3 = 1024
L3 = 512
NJ3 = LATENT_DIM // L3


def _dec_kernel(pre_ref, tau_ref, wd_ref, bd_ref, lat_ref, rec_ref):
    j = pl.program_id(1)
    pre = pre_ref[...]
    u = jax.lax.bitcast_convert_type(pre, jnp.int32) & jnp.int32(0x7FFFFFFF)
    lat = jnp.where(u >= tau_ref[:, 0:1], pre, 0.0)
    lat_ref[...] = lat

    partial = jax.lax.dot_general(
        lat.astype(jnp.bfloat16), wd_ref[...],
        (((1,), (1,)), ((), ())),
        preferred_element_type=jnp.float32)

    @pl.when(j == 0)
    def _init():
        rec_ref[...] = partial + bd_ref[...]

    @pl.when(j > 0)
    def _acc():
        rec_ref[...] = rec_ref[...] + partial


def _decode(pre, tau, Wd16, b_dec2d):
    return pl.pallas_call(
        _dec_kernel,
        grid=(N_TOKENS // R3, NJ3),
        in_specs=[
            pl.BlockSpec((R3, L3), lambda i, j: (i, j)),
            pl.BlockSpec((R3, 128), lambda i, j: (i, 0)),
            pl.BlockSpec((INPUT_DIM, L3), lambda i, j: (0, j)),
            pl.BlockSpec((1, INPUT_DIM), lambda i, j: (0, 0)),
        ],
        out_specs=[
            pl.BlockSpec((R3, L3), lambda i, j: (i, j)),
            pl.BlockSpec((R3, INPUT_DIM), lambda i, j: (i, 0)),
        ],
        out_shape=[
            jax.ShapeDtypeStruct((N_TOKENS, LATENT_DIM), jnp.float32),
            jax.ShapeDtypeStruct((N_TOKENS, INPUT_DIM), jnp.float32),
        ],
        compiler_params=pltpu.CompilerParams(
            dimension_semantics=("parallel", "arbitrary")),
    )(pre, tau, Wd16, b_dec2d)


@jax.jit
def kernel(x, W_enc, b_enc, W_dec, b_dec):
    x16 = x.astype(jnp.bfloat16)
    We16 = W_enc.astype(jnp.bfloat16)
    Wd16 = W_dec.astype(jnp.bfloat16)
    pre = _encode(x16, We16, b_enc.reshape(1, LATENT_DIM))
    tau = _thresholds(pre)
    latents, reconstructed = _decode(pre, tau, Wd16,
                                     b_dec.reshape(1, INPUT_DIM))
    return (reconstructed, latents, pre)
